# Initial kernel scaffold; baseline (speedup 1.0000x reference)
import jax
import jax.numpy as jnp
from jax.experimental import pallas as pl


def kernel(x, agg_orbs_inds, W1, b1, W2, b2, W3, b3):
    raise NotImplementedError("write your pallas kernel here")


# speedup vs baseline: 2.4662x; 2.4662x over previous
import numpy as np
import jax
import jax.numpy as jnp
from jax.experimental import pallas as pl
from jax.experimental.pallas import tpu as pltpu


def _orbit_count() -> int:
    ii, jj = np.meshgrid(np.arange(128), np.arange(128), indexing='ij')
    radius = np.sqrt(ii.astype(np.float64) ** 2 + jj.astype(np.float64) ** 2)
    return int(len(np.unique(np.round(radius))))


N_ORBS = _orbit_count()
O_PAD = 256
N_BLK = 512


def _gigp_kernel(ids_ref, x_ref, W1_ref, b1_ref, W2_ref, b2_ref, W3_ref,
                 b3_ref, out_ref, acc_ref):
    nb = pl.program_id(1)
    n_blocks = pl.num_programs(1)

    @pl.when(nb == 0)
    def _init():
        acc_ref[...] = jnp.zeros_like(acc_ref)

    ids = ids_ref[0]
    ot = jax.lax.broadcasted_iota(jnp.int32, (O_PAD, N_BLK), 0)
    onehot_t = (ot == ids).astype(jnp.float32)
    xb = x_ref[0]
    acc_ref[...] += jax.lax.dot_general(
        onehot_t, xb, (((1,), (1,)), ((), ())),
        preferred_element_type=jnp.float32)

    @pl.when(nb == n_blocks - 1)
    def _mlp():
        agg = acc_ref[...]
        h = jnp.maximum(
            jnp.dot(agg, W1_ref[...], preferred_element_type=jnp.float32)
            + b1_ref[...], 0.0)
        h = jnp.maximum(
            jnp.dot(h, W2_ref[...], preferred_element_type=jnp.float32)
            + b2_ref[...], 0.0)
        t = (jnp.dot(h, W3_ref[...], preferred_element_type=jnp.float32)
             + b3_ref[...])
        row = jax.lax.broadcasted_iota(jnp.int32, t.shape, 0)
        t = jnp.where(row < N_ORBS, t, 0.0)
        out_ref[0] = jnp.sum(t, axis=0, keepdims=True)


def kernel(x, agg_orbs_inds, W1, b1, W2, b2, W3, b3):
    B, C, N = x.shape
    n_blocks = N // N_BLK
    ids3 = agg_orbs_inds.reshape(n_blocks, 1, N_BLK)
    out = pl.pallas_call(
        _gigp_kernel,
        grid=(B, n_blocks),
        in_specs=[
            pl.BlockSpec((1, 1, N_BLK), lambda b, nb: (nb, 0, 0)),
            pl.BlockSpec((1, C, N_BLK), lambda b, nb: (b, 0, nb)),
            pl.BlockSpec(W1.shape, lambda b, nb: (0, 0)),
            pl.BlockSpec((1, b1.shape[0]), lambda b, nb: (0, 0)),
            pl.BlockSpec(W2.shape, lambda b, nb: (0, 0)),
            pl.BlockSpec((1, b2.shape[0]), lambda b, nb: (0, 0)),
            pl.BlockSpec(W3.shape, lambda b, nb: (0, 0)),
            pl.BlockSpec((1, b3.shape[0]), lambda b, nb: (0, 0)),
        ],
        out_specs=pl.BlockSpec((1, 1, W3.shape[1]), lambda b, nb: (b, 0, 0)),
        out_shape=jax.ShapeDtypeStruct((B, 1, W3.shape[1]), jnp.float32),
        scratch_shapes=[pltpu.VMEM((O_PAD, C), jnp.float32)],
    )(ids3, x, W1, b1.reshape(1, -1), W2, b2.reshape(1, -1), W3,
      b3.reshape(1, -1))
    return out.reshape(B, W3.shape[1])
